# baseline (speedup 1.0000x reference)
import jax
import jax.numpy as jnp
from jax.experimental import pallas as pl
from jax.experimental.pallas import tpu as pltpu

_CHUNKS = 4
_G = 128


def _vq_kernel(z_hbm, dic_ref, out_hbm, z_vmem, out_vmem, in_sem, out_sem):
    dic = dic_ref[...]
    k, d = dic.shape
    n = z_hbm.shape[0]
    c = n // _CHUNKS

    def in_copy(h):
        return pltpu.make_async_copy(
            z_hbm.at[pl.ds(h * c, c), :], z_vmem.at[h % 2], in_sem.at[h % 2])

    def out_copy(h):
        return pltpu.make_async_copy(
            out_vmem.at[h % 2], out_hbm.at[pl.ds(h * c, c), :],
            out_sem.at[h % 2])

    in_copy(0).start()
    in_copy(1).start()

    ones = jnp.ones((1, d), jnp.float32)
    d2 = jax.lax.dot_general(
        ones, dic * dic, (((1,), (1,)), ((), ())),
        precision=jax.lax.Precision.HIGHEST,
        preferred_element_type=jnp.float32)

    for h in range(_CHUNKS):
        slot = h % 2
        in_copy(h).wait()
        z = z_vmem[slot]
        cross = jax.lax.dot_general(
            z, dic, (((1,), (1,)), ((), ())),
            precision=jax.lax.Precision.DEFAULT,
            preferred_element_type=jnp.float32)
        z2 = jnp.sum(z * z, axis=1, keepdims=True)
        if h + 2 < _CHUNKS:
            in_copy(h + 2).start()
        giota = jax.lax.broadcasted_iota(
            jnp.int32, (c, _G), 1).astype(jnp.float32)
        vacc = (d2[:, :_G] + z2) - 2.0 * cross[:, :_G]
        iacc = giota
        for g in range(1, k // _G):
            dist_g = (d2[:, g * _G:(g + 1) * _G] + z2) \
                - 2.0 * cross[:, g * _G:(g + 1) * _G]
            lt = dist_g < vacc
            iacc = jnp.where(lt, giota + float(g * _G), iacc)
            vacc = jnp.minimum(vacc, dist_g)
        minval = jnp.min(vacc, axis=1, keepdims=True)
        idx = jnp.min(jnp.where(vacc == minval, iacc, float(k)), axis=1,
                      keepdims=True)
        iota = jax.lax.broadcasted_iota(
            jnp.int32, (c, k), 1).astype(jnp.float32)
        onehot = (iota == idx).astype(jnp.bfloat16)
        if h >= 2:
            out_copy(h - 2).wait()
        out_vmem[slot] = jax.lax.dot_general(
            onehot, dic, (((1,), (0,)), ((), ())),
            precision=jax.lax.Precision.DEFAULT,
            preferred_element_type=jnp.float32)
        out_copy(h).start()

    for h in range(_CHUNKS - 2, _CHUNKS):
        out_copy(h).wait()


def kernel(ze, dictionary):
    b, t, d = ze.shape
    n = b * t
    k = dictionary.shape[0]
    z = ze.reshape(n, d)
    c = n // _CHUNKS
    out = pl.pallas_call(
        _vq_kernel,
        grid=(1,),
        in_specs=[
            pl.BlockSpec(memory_space=pl.ANY),
            pl.BlockSpec((k, d), lambda i: (0, 0)),
        ],
        out_specs=pl.BlockSpec(memory_space=pl.ANY),
        out_shape=jax.ShapeDtypeStruct((n, d), jnp.float32),
        scratch_shapes=[
            pltpu.VMEM((2, c, d), jnp.float32),
            pltpu.VMEM((2, c, d), jnp.float32),
            pltpu.SemaphoreType.DMA((2,)),
            pltpu.SemaphoreType.DMA((2,)),
        ],
    )(z, dictionary)
    return out.reshape(b, t, d)


# speedup vs baseline: 1.2511x; 1.2511x over previous
import jax
import jax.numpy as jnp
from jax.experimental import pallas as pl

_CHUNKS = 4
_G = 128


def _vq_kernel(z_ref, dic_ref, out_ref):
    dic = dic_ref[...]
    k, d = dic.shape
    n = z_ref.shape[0]
    ones = jnp.ones((1, d), jnp.float32)
    d2 = jax.lax.dot_general(
        ones, dic * dic, (((1,), (1,)), ((), ())),
        precision=jax.lax.Precision.HIGHEST,
        preferred_element_type=jnp.float32)
    c = n // _CHUNKS
    dic_bf = dic.astype(jnp.bfloat16)
    giota = jax.lax.broadcasted_iota(
        jnp.int32, (c, _G), 1).astype(jnp.float32)
    iota = jax.lax.broadcasted_iota(
        jnp.int32, (c, k), 1).astype(jnp.float32)
    for h in range(_CHUNKS):
        z = z_ref[h * c:(h + 1) * c, :]
        cross = jax.lax.dot_general(
            z.astype(jnp.bfloat16), dic_bf, (((1,), (1,)), ((), ())),
            precision=jax.lax.Precision.DEFAULT,
            preferred_element_type=jnp.float32)
        z2 = jnp.sum(z * z, axis=1, keepdims=True)
        vacc = (d2[:, :_G] + z2) - 2.0 * cross[:, :_G]
        iacc = giota
        for g in range(1, k // _G):
            dist_g = (d2[:, g * _G:(g + 1) * _G] + z2) \
                - 2.0 * cross[:, g * _G:(g + 1) * _G]
            lt = dist_g < vacc
            iacc = jnp.where(lt, giota + float(g * _G), iacc)
            vacc = jnp.minimum(vacc, dist_g)
        minval = jnp.min(vacc, axis=1, keepdims=True)
        idx = jnp.min(jnp.where(vacc == minval, iacc, float(k)), axis=1,
                      keepdims=True)
        onehot = (iota == idx).astype(jnp.bfloat16)
        out_ref[h * c:(h + 1) * c, :] = jax.lax.dot_general(
            onehot, dic_bf, (((1,), (0,)), ((), ())),
            precision=jax.lax.Precision.DEFAULT,
            preferred_element_type=jnp.float32)


def kernel(ze, dictionary):
    b, t, d = ze.shape
    n = b * t
    k = dictionary.shape[0]
    z = ze.reshape(n, d)
    out = pl.pallas_call(
        _vq_kernel,
        grid=(1,),
        in_specs=[
            pl.BlockSpec((n, d), lambda i: (0, 0)),
            pl.BlockSpec((k, d), lambda i: (0, 0)),
        ],
        out_specs=pl.BlockSpec((n, d), lambda i: (0, 0)),
        out_shape=jax.ShapeDtypeStruct((n, d), jnp.float32),
    )(z, dictionary)
    return out.reshape(b, t, d)
